# baseline (speedup 1.0000x reference)
import functools

import jax
import jax.numpy as jnp
from jax import lax
from jax.experimental import pallas as pl
from jax.experimental.pallas import tpu as pltpu
from jax.experimental.pallas import tpu_sc as plsc

_NC, _NS, _L = 2, 16, 16
_NW = _NC * _NS
_SEG = 8


def _unpool_planes(vals_t, mask_t, n, p):
    rows = vals_t.shape[0] // n
    assert rows % (2 * _NW) == 0
    items = rows // _NW
    items2 = items // 2
    chout = p // _SEG
    chin = n // _SEG
    sun = 7
    zun = 8
    assert chin % (_L * sun) == 0 and chout % (_L * zun) == 0

    mesh = plsc.VectorSubcoreMesh(
        core_axis_name="c", subcore_axis_name="s",
        num_cores=_NC, num_subcores=_NS,
    )

    @functools.partial(
        pl.kernel,
        out_type=jax.ShapeDtypeStruct((rows * p,), jnp.float32),
        mesh=mesh,
        compiler_params=pltpu.CompilerParams(needs_layout_passes=False),
        scratch_types=[
            pltpu.VMEM((p,), jnp.float32),
            pltpu.VMEM((p,), jnp.float32),
            pltpu.VMEM((chin,), jnp.float32),
            pltpu.VMEM((chin,), jnp.float32),
            pltpu.VMEM((chin,), jnp.int32),
            pltpu.VMEM((chin,), jnp.int32),
            pltpu.SemaphoreType.DMA,
            pltpu.SemaphoreType.DMA,
            pltpu.SemaphoreType.DMA,
            pltpu.SemaphoreType.DMA,
        ],
    )
    def k(vals_hbm, mask_hbm, out_hbm,
          acc_a, acc_b, vin_a, vin_b, min_a, min_b, s_ia, s_ib, s_oa, s_ob):
        wid = lax.axis_index("s") * _NC + lax.axis_index("c")
        inbufs = ((vin_a, min_a, s_ia), (vin_b, min_b, s_ib))

        def start_in(row, seg, which):
            vbuf, mbuf, sem = inbufs[which]
            src_v = vals_hbm.at[pl.ds(row * n + seg * chin, chin)]
            src_m = mask_hbm.at[pl.ds(row * n + seg * chin, chin)]
            pltpu.make_async_copy(src_v, vbuf, sem).start()
            pltpu.make_async_copy(src_m, mbuf, sem).start()

        def wait_in(which):
            vbuf, mbuf, sem = inbufs[which]
            pltpu.make_async_copy(vals_hbm.at[pl.ds(0, chin)], vbuf, sem).wait()
            pltpu.make_async_copy(mask_hbm.at[pl.ds(0, chin)], mbuf, sem).wait()

        def scat_seg(which, acc):
            vbuf, mbuf, _ = inbufs[which]

            def sb(i, c):
                base = i * (_L * sun)
                for u in range(sun):
                    off = base + u * _L
                    m = mbuf[pl.ds(off, _L)]
                    x = (m >> 6).astype(jnp.float32)
                    idx = ((x + 0.5) * (1.0 / 3.0)).astype(jnp.int32)
                    v = vbuf[pl.ds(off, _L)]
                    plsc.addupdate_scatter(acc, [idx], v)
                return c

            lax.fori_loop(0, chin // (_L * sun), sb, 0)

        zv = jnp.zeros((_L,), jnp.float32)

        def zero_chunk(acc, seg):
            def zb(i, c):
                off = seg * chout + i * (_L * zun)
                for u in range(zun):
                    acc[pl.ds(off + u * _L, _L)] = zv
                return c

            lax.fori_loop(0, chout // (_L * zun), zb, 0)

        def out_chunk_copy(acc, sem, row_prev, seg):
            return pltpu.make_async_copy(
                acc.at[pl.ds(seg * chout, chout)],
                out_hbm.at[pl.ds(row_prev * p + seg * chout, chout)],
                sem,
            )

        def run_item(row, row_prev, acc_cur, acc_prev, sem_prev,
                     drain_pred, prefetch_pred):
            for seg in range(_SEG):
                @pl.when(drain_pred)
                def _():
                    out_chunk_copy(acc_prev, sem_prev, row_prev, seg).start()

                wait_in(seg % 2)
                if seg + 1 < _SEG:
                    start_in(row, seg + 1, (seg + 1) % 2)
                else:
                    @pl.when(prefetch_pred)
                    def _():
                        start_in(row + _NW, 0, 0)

                scat_seg(seg % 2, acc_cur)

                @pl.when(drain_pred)
                def _():
                    out_chunk_copy(acc_prev, sem_prev, row_prev, seg).wait()
                    zero_chunk(acc_prev, seg)

        start_in(wid, 0, 0)
        for seg in range(_SEG):
            zero_chunk(acc_a, seg)
            zero_chunk(acc_b, seg)

        true_p = wid >= 0

        def body(i2, c):
            row_a = (2 * i2) * _NW + wid
            row_b = row_a + _NW
            run_item(row_a, row_a - _NW, acc_a, acc_b, s_ob,
                     drain_pred=i2 > 0, prefetch_pred=true_p)
            run_item(row_b, row_a, acc_b, acc_a, s_oa,
                     drain_pred=true_p, prefetch_pred=i2 + 1 < items2)
            return c

        lax.fori_loop(0, items2, body, 0)

        last_row = (items - 1) * _NW + wid
        for seg in range(_SEG):
            out_chunk_copy(acc_b, s_ob, last_row, seg).start()
        for seg in range(_SEG):
            out_chunk_copy(acc_b, s_ob, last_row, seg).wait()

    return k(vals_t, mask_t)


def kernel(updates, mask):
    b, h, w, c = updates.shape
    n = h * w
    hout, wout = 2 * h, 2 * w
    p = hout * wout
    assert c == 192
    vals_t = updates.reshape(b, n, c).transpose(0, 2, 1).reshape(b * c * n)
    mask_t = mask.astype(jnp.int32).reshape(b, n, c).transpose(0, 2, 1)
    mask_t = mask_t.reshape(b * c * n)
    out_t = _unpool_planes(vals_t, mask_t, n, p)
    return out_t.reshape(b, c, hout, wout).transpose(0, 2, 3, 1)


# speedup vs baseline: 1.2329x; 1.2329x over previous
import functools

import jax
import jax.numpy as jnp
from jax import lax
from jax.experimental import pallas as pl
from jax.experimental.pallas import tpu as pltpu
from jax.experimental.pallas import tpu_sc as plsc

_NC, _NS, _L = 2, 16, 16
_NW = _NC * _NS


def _stage_in(updates, mask, n, hb):
    b, h, w, c = updates.shape
    nt = hb * w
    grid = (b, h // hb)

    def tin(u_ref, m_ref, v_ref, i_ref):
        u = u_ref[0].reshape(nt, c)
        m = m_ref[0].reshape(nt, c)
        v_ref[...] = u.T
        i_ref[...] = (m >> 6).T // 3

    return pl.pallas_call(
        tin,
        grid=grid,
        in_specs=[
            pl.BlockSpec((1, hb, w, c), lambda i, j: (i, j, 0, 0)),
            pl.BlockSpec((1, hb, w, c), lambda i, j: (i, j, 0, 0)),
        ],
        out_specs=[
            pl.BlockSpec((c, nt), lambda i, j: (i, j)),
            pl.BlockSpec((c, nt), lambda i, j: (i, j)),
        ],
        out_shape=[
            jax.ShapeDtypeStruct((b * c, n), jnp.float32),
            jax.ShapeDtypeStruct((b * c, n), jnp.int32),
        ],
    )(updates, mask.astype(jnp.int32))


def _stage_out(out_t, b, c, hout, wout, pt):
    p = hout * wout
    rows_per_blk = pt // wout

    def tout(t_ref, o_ref):
        o_ref[...] = t_ref[...].T.reshape(1, rows_per_blk, wout, c)

    return pl.pallas_call(
        tout,
        grid=(b, p // pt),
        in_specs=[pl.BlockSpec((c, pt), lambda i, j: (i, j))],
        out_specs=pl.BlockSpec(
            (1, rows_per_blk, wout, c), lambda i, j: (i, j, 0, 0)
        ),
        out_shape=jax.ShapeDtypeStruct((b, hout, wout, c), jnp.float32),
    )(out_t)


def _unpool_planes(vals_t, idx_t, n, p):
    rows = vals_t.shape[0]
    assert rows % _NW == 0
    items = rows // _NW
    chunks = 8
    ch = p // chunks
    zun = 8
    sun = 8
    assert ch % (_L * zun) == 0 and n % (_L * sun) == 0

    mesh = plsc.VectorSubcoreMesh(
        core_axis_name="c", subcore_axis_name="s",
        num_cores=_NC, num_subcores=_NS,
    )

    @functools.partial(
        pl.kernel,
        out_type=jax.ShapeDtypeStruct((rows, p), jnp.float32),
        mesh=mesh,
        compiler_params=pltpu.CompilerParams(needs_layout_passes=False),
        scratch_types=[
            pltpu.VMEM((n,), jnp.float32),
            pltpu.VMEM((n,), jnp.int32),
            pltpu.VMEM((n,), jnp.float32),
            pltpu.VMEM((n,), jnp.int32),
            pltpu.VMEM((p,), jnp.float32),
            pltpu.SemaphoreType.DMA,
            pltpu.SemaphoreType.DMA,
            pltpu.SemaphoreType.DMA,
            pltpu.SemaphoreType.DMA,
        ],
    )
    def k(vals_hbm, idx_hbm, out_hbm,
          va, ia, vb, ib, acc, s_a, s_b, so0, so1):
        wid = lax.axis_index("s") * _NC + lax.axis_index("c")
        inbufs = ((va, ia, s_a), (vb, ib, s_b))

        def start_in(row, which):
            vbuf, ibuf, sem = inbufs[which]
            pltpu.make_async_copy(vals_hbm.at[row], vbuf, sem).start()
            pltpu.make_async_copy(idx_hbm.at[row], ibuf, sem).start()

        def wait_in(which):
            vbuf, ibuf, sem = inbufs[which]
            pltpu.make_async_copy(vals_hbm.at[0], vbuf, sem).wait()
            pltpu.make_async_copy(idx_hbm.at[0], ibuf, sem).wait()

        zv = jnp.zeros((_L,), jnp.float32)

        def zero_chunk(seg):
            def zb(i, c_):
                off = seg * ch + i * (_L * zun)
                for u in range(zun):
                    acc[pl.ds(off + u * _L, _L)] = zv
                return c_

            lax.fori_loop(0, ch // (_L * zun), zb, 0)

        def scatter_item(which):
            vbuf, ibuf, _ = inbufs[which]

            def sb(i, c_):
                base = i * (_L * sun)
                for u in range(sun):
                    off = base + u * _L
                    idx = ibuf[pl.ds(off, _L)]
                    v = vbuf[pl.ds(off, _L)]
                    plsc.addupdate_scatter(acc, [idx], v)
                return c_

            lax.fori_loop(0, n // (_L * sun), sb, 0)

        def drain_item(row):
            sems = (so0, so1)
            cps = []
            for j in range(chunks):
                cp = pltpu.make_async_copy(
                    acc.at[pl.ds(j * ch, ch)],
                    out_hbm.at[row, pl.ds(j * ch, ch)],
                    sems[j % 2],
                )
                cp.start()
                if j >= 1:
                    cps[j - 1].wait()
                    zero_chunk(j - 1)
                cps.append(cp)
            cps[-1].wait()
            zero_chunk(chunks - 1)

        start_in(wid, 0)
        for seg in range(chunks):
            zero_chunk(seg)

        def body(i2, c_):
            row_a = (2 * i2) * _NW + wid
            row_b = row_a + _NW
            start_in(row_b, 1)
            wait_in(0)
            scatter_item(0)
            drain_item(row_a)

            @pl.when(i2 + 1 < items // 2)
            def _():
                start_in(row_b + _NW, 0)

            wait_in(1)
            scatter_item(1)
            drain_item(row_b)
            return c_

        lax.fori_loop(0, items // 2, body, 0)

    return k(vals_t, idx_t)


def kernel(updates, mask):
    b, h, w, c = updates.shape
    n = h * w
    hout, wout = 2 * h, 2 * w
    p = hout * wout
    assert c == 192
    vals_t, idx_t = _stage_in(updates, mask, n, hb=16)
    out_t = _unpool_planes(vals_t, idx_t, n, p)
    return _stage_out(out_t, b, c, hout, wout, pt=1792)
